# baseline (speedup 1.0000x reference)
import functools

import jax
import jax.numpy as jnp
from jax import lax
from jax.experimental import pallas as pl
from jax.experimental.pallas import tpu as pltpu
from jax.experimental.pallas import tpu_sc as plsc

VOCAB = 1682
EMBED_DIM = 32
BATCH = 16384

_info = plsc.get_sparse_core_info()
_NC, _NS, _L = _info.num_cores, _info.num_subcores, _info.num_lanes
_NW = _NC * _NS
_B_PER_W = BATCH // _NW
_GCHUNK = 128
_NG = _B_PER_W // _GCHUNK


def _make_sc_gather():
    mesh = plsc.VectorSubcoreMesh(core_axis_name="c", subcore_axis_name="s")

    @functools.partial(
        pl.kernel,
        mesh=mesh,
        out_type=jax.ShapeDtypeStruct((BATCH, EMBED_DIM), jnp.float32),
        scratch_types=[
            pltpu.VMEM((_NG, _GCHUNK), jnp.int32),
            pltpu.VMEM((_B_PER_W, EMBED_DIM), jnp.float32),
            [pltpu.SemaphoreType.DMA] * _NG,
            pltpu.SemaphoreType.DMA,
        ],
        compiler_params=pltpu.CompilerParams(
            use_tc_tiling_on_sc=False,
            disable_bounds_checks=True,
            disable_semaphore_checks=True,
        ),
    )
    def sc_gather(ids_hbm, table_hbm, out_hbm, idx_v, rows_v, gsem, osem):
        wid = lax.axis_index("s") * _NC + lax.axis_index("c")
        base = wid * _B_PER_W

        pltpu.sync_copy(ids_hbm.at[wid], idx_v)

        for j in range(_NG):
            for i in range(_GCHUNK // _L):
                sl = pl.ds(i * _L, _L)
                idx_v[j, sl] = idx_v[j, sl] + 1

        gathers = [
            pltpu.async_copy(
                table_hbm.at[idx_v.at[j]],
                rows_v.at[pl.ds(j * _GCHUNK, _GCHUNK)],
                gsem[j],
            )
            for j in range(_NG)
        ]
        writes = []
        for j in range(_NG):
            gathers[j].wait()
            writes.append(
                pltpu.async_copy(
                    rows_v.at[pl.ds(j * _GCHUNK, _GCHUNK)],
                    out_hbm.at[pl.ds(base + j * _GCHUNK, _GCHUNK)],
                    osem,
                )
            )
        for c in writes:
            c.wait()

    return sc_gather


_sc_gather = _make_sc_gather()


def kernel(movie_id, table):
    ids = movie_id.reshape(_NW, _NG, _GCHUNK)
    return _sc_gather(ids, table)


# speedup vs baseline: 1.0306x; 1.0306x over previous
import functools

import jax
import jax.numpy as jnp
from jax import lax
from jax.experimental import pallas as pl
from jax.experimental.pallas import tpu as pltpu
from jax.experimental.pallas import tpu_sc as plsc

VOCAB = 1682
EMBED_DIM = 32
BATCH = 16384

_info = plsc.get_sparse_core_info()
_NC, _NS, _L = 1, _info.num_subcores, _info.num_lanes
_NW = _NC * _NS
_B_PER_W = BATCH // _NW
_GCHUNK = 128
_NG = _B_PER_W // _GCHUNK


def _make_sc_gather():
    mesh = plsc.VectorSubcoreMesh(
        core_axis_name="c", subcore_axis_name="s", num_cores=_NC
    )

    @functools.partial(
        pl.kernel,
        mesh=mesh,
        out_type=jax.ShapeDtypeStruct((BATCH, EMBED_DIM), jnp.float32),
        scratch_types=[
            pltpu.VMEM((_NG, _GCHUNK), jnp.int32),
            pltpu.VMEM((_B_PER_W, EMBED_DIM), jnp.float32),
            [pltpu.SemaphoreType.DMA] * _NG,
            pltpu.SemaphoreType.DMA,
        ],
        compiler_params=pltpu.CompilerParams(
            use_tc_tiling_on_sc=False,
            disable_bounds_checks=True,
            disable_semaphore_checks=True,
        ),
    )
    def sc_gather(ids_hbm, table_hbm, out_hbm, idx_v, rows_v, gsem, osem):
        wid = lax.axis_index("s") * _NC + lax.axis_index("c")
        base = wid * _B_PER_W

        pltpu.sync_copy(ids_hbm.at[wid], idx_v)

        for j in range(_NG):
            for i in range(_GCHUNK // _L):
                sl = pl.ds(i * _L, _L)
                idx_v[j, sl] = idx_v[j, sl] + 1

        gathers = [
            pltpu.async_copy(
                table_hbm.at[idx_v.at[j]],
                rows_v.at[pl.ds(j * _GCHUNK, _GCHUNK)],
                gsem[j],
            )
            for j in range(_NG)
        ]
        writes = []
        for j in range(_NG):
            gathers[j].wait()
            writes.append(
                pltpu.async_copy(
                    rows_v.at[pl.ds(j * _GCHUNK, _GCHUNK)],
                    out_hbm.at[pl.ds(base + j * _GCHUNK, _GCHUNK)],
                    osem,
                )
            )
        for c in writes:
            c.wait()

    return sc_gather


_sc_gather = _make_sc_gather()


def kernel(movie_id, table):
    ids = movie_id.reshape(_NW, _NG, _GCHUNK)
    return _sc_gather(ids, table)


# speedup vs baseline: 1.0369x; 1.0061x over previous
import functools

import jax
import jax.numpy as jnp
from jax import lax
from jax.experimental import pallas as pl
from jax.experimental.pallas import tpu as pltpu
from jax.experimental.pallas import tpu_sc as plsc

VOCAB = 1682
EMBED_DIM = 32
BATCH = 16384

_info = plsc.get_sparse_core_info()
_NC, _NS, _L = 1, _info.num_subcores, _info.num_lanes
_NW = _NC * _NS
_B_PER_W = BATCH // _NW
_GCHUNK = 128
_NG = _B_PER_W // _GCHUNK


def _make_sc_gather():
    mesh = plsc.VectorSubcoreMesh(
        core_axis_name="c", subcore_axis_name="s", num_cores=_NC
    )

    @functools.partial(
        pl.kernel,
        mesh=mesh,
        out_type=jax.ShapeDtypeStruct((BATCH, EMBED_DIM), jnp.float32),
        scratch_types=[
            pltpu.VMEM((_NG, _GCHUNK), jnp.int32),
            pltpu.VMEM((_B_PER_W, EMBED_DIM), jnp.float32),
            [pltpu.SemaphoreType.DMA] * _NG,
            pltpu.SemaphoreType.DMA,
        ],
        compiler_params=pltpu.CompilerParams(
            use_tc_tiling_on_sc=False,
            disable_bounds_checks=True,
            disable_semaphore_checks=True,
        ),
    )
    def sc_gather(ids_hbm, table_hbm, out_hbm, idx_v, rows_v, gsem, osem):
        wid = lax.axis_index("s") * _NC + lax.axis_index("c")
        base = wid * _B_PER_W

        pltpu.sync_copy(ids_hbm.at[wid], idx_v)

        shifted = table_hbm.at[pl.ds(1, VOCAB)]

        gathers = [
            pltpu.async_copy(
                shifted.at[idx_v.at[j]],
                rows_v.at[pl.ds(j * _GCHUNK, _GCHUNK)],
                gsem[j],
            )
            for j in range(_NG)
        ]
        writes = []
        for j in range(_NG):
            gathers[j].wait()
            writes.append(
                pltpu.async_copy(
                    rows_v.at[pl.ds(j * _GCHUNK, _GCHUNK)],
                    out_hbm.at[pl.ds(base + j * _GCHUNK, _GCHUNK)],
                    osem,
                )
            )
        for c in writes:
            c.wait()

    return sc_gather


_sc_gather = _make_sc_gather()


def kernel(movie_id, table):
    ids = movie_id.reshape(_NW, _NG, _GCHUNK)
    return _sc_gather(ids, table)
